# baseline (speedup 1.0000x reference)
import functools

import jax
import jax.numpy as jnp
from jax import lax
from jax.experimental import pallas as pl
from jax.experimental.pallas import tpu as pltpu
from jax.experimental.pallas import tpu_sc as plsc

DIM_IN = 64
NUM_INDS = 16
N_BINS = 64
B = 8
N = 65536

NC = 2
NS = 16
NW = NC * NS

N_ROWS = B * NUM_INDS
ROWS_PER_W = N_ROWS // NW
CHUNK_VALS = 32768
CHUNKS_PER_ROW = N // CHUNK_VALS
N_CHUNKS = ROWS_PER_W * CHUNKS_PER_ROW
HIST_W = 81
OUT_W = 80
UNROLL = 16
NBLK = 4096


def _pos_body(iw_ref, x_ref, out_ref):
    s = lax.dot_general(iw_ref[...], x_ref[0],
                        (((1,), (0,)), ((), ())),
                        preferred_element_type=jnp.float32)
    v = jax.nn.sigmoid(s)
    out_ref[0] = jnp.clip(v * float(N_BINS) - 0.5, 0.0, float(N_BINS - 1))


def _compute_pos(Xt, Iw):
    grid = (B, N // NBLK)
    return pl.pallas_call(
        _pos_body,
        grid=grid,
        in_specs=[
            pl.BlockSpec((NUM_INDS, DIM_IN), lambda b, j: (0, 0)),
            pl.BlockSpec((1, DIM_IN, NBLK), lambda b, j: (b, 0, j)),
        ],
        out_specs=pl.BlockSpec((1, NUM_INDS, NBLK), lambda b, j: (b, 0, j)),
        out_shape=jax.ShapeDtypeStruct((B, NUM_INDS, N), jnp.float32),
    )(Iw, Xt)


def _hist_body(pos_hbm, out_hbm, buf0, buf1, hist, stage, sem0, sem1):
    wid = lax.axis_index("s") * NC + lax.axis_index("c")
    base = wid * (ROWS_PER_W * N)

    zeros16 = jnp.zeros((16,), jnp.float32)
    lane_iota = lax.iota(jnp.int32, 16)
    lane_base = lane_iota * HIST_W

    bufs = [buf0, buf1]
    sems = [sem0, sem1]

    def _copy(c):
        return pltpu.make_async_copy(
            pos_hbm.at[pl.ds(base + c * CHUNK_VALS, CHUNK_VALS)],
            bufs[c % 2], sems[c % 2],
        )

    _copy(0).start()
    for c in range(N_CHUNKS):
        if c + 1 < N_CHUNKS:
            _copy(c + 1).start()
        if c % CHUNKS_PER_ROW == 0:
            for i in range(16 * HIST_W // 16):
                hist[pl.ds(i * 16, 16)] = zeros16
        _copy(c).wait()
        buf = bufs[c % 2]

        @plsc.parallel_loop(0, CHUNK_VALS // 16, 1, unroll=UNROLL)
        def _vec(r):
            v = buf[pl.ds(r * 16, 16)]
            i0 = v.astype(jnp.int32)
            frac = v - i0.astype(jnp.float32)
            idx0 = lane_base + i0
            plsc.addupdate_scatter(hist, [idx0], 1.0 - frac)
            plsc.addupdate_scatter(hist, [idx0 + 1], frac)

        if c % CHUNKS_PER_ROW == CHUNKS_PER_ROW - 1:
            row = c // CHUNKS_PER_ROW
            for g in range(OUT_W // 16):
                acc = zeros16
                for l in range(16):
                    acc = acc + plsc.load_gather(
                        hist, [lane_iota + jnp.int32(l * HIST_W + g * 16)])
                stage[pl.ds(row * OUT_W + g * 16, 16)] = acc

    pltpu.sync_copy(stage, out_hbm.at[pl.ds(wid * (ROWS_PER_W * OUT_W),
                                            ROWS_PER_W * OUT_W)])


_hist_call = functools.partial(
    pl.kernel,
    out_type=jax.ShapeDtypeStruct((N_ROWS * OUT_W,), jnp.float32),
    mesh=plsc.VectorSubcoreMesh(core_axis_name="c", subcore_axis_name="s"),
    scratch_types=[
        pltpu.VMEM((CHUNK_VALS,), jnp.float32),
        pltpu.VMEM((CHUNK_VALS,), jnp.float32),
        pltpu.VMEM((16 * HIST_W,), jnp.float32),
        pltpu.VMEM((ROWS_PER_W * OUT_W,), jnp.float32),
        pltpu.SemaphoreType.DMA,
        pltpu.SemaphoreType.DMA,
    ],
    compiler_params=pltpu.CompilerParams(needs_layout_passes=False),
)(_hist_body)


def kernel(X, I):
    Xt = X.transpose(0, 2, 1)
    Iw = I[0]
    pos = _compute_pos(Xt, Iw)
    hist = _hist_call(pos.reshape(B * NUM_INDS * N))
    hist = hist.reshape(B, NUM_INDS, OUT_W)[:, :, :N_BINS] * (1.0 / N)
    return hist.reshape(B, NUM_INDS * N_BINS)
